# baseline (speedup 1.0000x reference)
import functools

import jax
import jax.numpy as jnp
from jax import lax
from jax.experimental import pallas as pl
from jax.experimental.pallas import tpu as pltpu
from jax.experimental.pallas import tpu_sc as plsc

NUM_ITEMS = 1000000
LATENT = 64
BATCH = 16384

NC = 2
NS = 16
NW = NC * NS
B_PER_W = BATCH // NW

_BLK = 51200


def _mv_body(xT_ref, w_ref, b_ref, o_ref):
  x = xT_ref[...]
  w = w_ref[...]
  o_ref[...] = jnp.sum(x * w, axis=0) + b_ref[0]


def _matvec_stage(tableT, W, b):
  grid = (pl.cdiv(NUM_ITEMS, _BLK),)
  return pl.pallas_call(
      _mv_body,
      grid=grid,
      in_specs=[
          pl.BlockSpec((LATENT, _BLK), lambda i: (0, i)),
          pl.BlockSpec((LATENT, 1), lambda i: (0, 0)),
          pl.BlockSpec(memory_space=pltpu.SMEM),
      ],
      out_specs=pl.BlockSpec((_BLK,), lambda i: (i,)),
      out_shape=jax.ShapeDtypeStruct((NUM_ITEMS,), jnp.float32),
  )(tableT, W.reshape(LATENT, 1), b)


def _make_select():
  mesh = plsc.VectorSubcoreMesh(
      core_axis_name="c", subcore_axis_name="s", num_cores=NC,
      num_subcores=NS)

  @functools.partial(
      pl.kernel,
      mesh=mesh,
      out_type=jax.ShapeDtypeStruct((BATCH,), jnp.float32),
      scratch_types=[
          pltpu.VMEM((B_PER_W,), jnp.int32),
          pltpu.VMEM((B_PER_W,), jnp.float32),
          pltpu.SemaphoreType.DMA,
      ],
  )
  def select_k(idx_hbm, s_hbm, out_hbm, idx_v, g_v, sem):
    cid = lax.axis_index("c")
    sid = lax.axis_index("s")
    wid = sid * NC + cid
    base = wid * B_PER_W
    pltpu.sync_copy(idx_hbm.at[pl.ds(base, B_PER_W)], idx_v)
    pltpu.async_copy(s_hbm.at[idx_v], g_v, sem).wait()
    for g in range(B_PER_W // 16):
      v = g_v[pl.ds(g * 16, 16)]
      r = 1.0 / (1.0 + jnp.exp(-v))
      g_v[pl.ds(g * 16, 16)] = r
    pltpu.sync_copy(g_v, out_hbm.at[pl.ds(base, B_PER_W)])

  return select_k


_select = _make_select()


@jax.jit
def kernel(item_indices, emb_table, W, b):
  idx0 = (item_indices - 1).astype(jnp.int32)
  s = _matvec_stage(emb_table.T, W, b)
  return _select(idx0, s)


# speedup vs baseline: 1.0113x; 1.0113x over previous
import functools

import jax
import jax.numpy as jnp
from jax import lax
from jax.experimental import pallas as pl
from jax.experimental.pallas import tpu as pltpu
from jax.experimental.pallas import tpu_sc as plsc

NUM_ITEMS = 1000000
LATENT = 64
BATCH = 16384

NC = 2
NS = 16
NW = NC * NS
B_PER_W = BATCH // NW

_BLK = 40960


def _mv_body(xT_ref, w_ref, b_ref, o_ref):
  x = xT_ref[...]
  w = w_ref[...]
  o_ref[...] = jnp.sum(x * w, axis=0) + b_ref[0]


def _matvec_stage(tableT, W, b):
  grid = (pl.cdiv(NUM_ITEMS, _BLK),)
  return pl.pallas_call(
      _mv_body,
      grid=grid,
      in_specs=[
          pl.BlockSpec((LATENT, _BLK), lambda i: (0, i)),
          pl.BlockSpec((LATENT, 1), lambda i: (0, 0)),
          pl.BlockSpec(memory_space=pltpu.SMEM),
      ],
      out_specs=pl.BlockSpec((_BLK,), lambda i: (i,)),
      out_shape=jax.ShapeDtypeStruct((NUM_ITEMS,), jnp.float32),
  )(tableT, W.reshape(LATENT, 1), b)


def _make_select():
  mesh = plsc.VectorSubcoreMesh(
      core_axis_name="c", subcore_axis_name="s", num_cores=NC,
      num_subcores=NS)

  @functools.partial(
      pl.kernel,
      mesh=mesh,
      out_type=jax.ShapeDtypeStruct((BATCH,), jnp.float32),
      scratch_types=[
          pltpu.VMEM((B_PER_W,), jnp.int32),
          pltpu.VMEM((B_PER_W,), jnp.float32),
          pltpu.SemaphoreType.DMA,
      ],
  )
  def select_k(idx_hbm, s_hbm, out_hbm, idx_v, g_v, sem):
    cid = lax.axis_index("c")
    sid = lax.axis_index("s")
    wid = sid * NC + cid
    base = wid * B_PER_W
    pltpu.sync_copy(idx_hbm.at[pl.ds(base, B_PER_W)], idx_v)
    pltpu.async_copy(s_hbm.at[idx_v], g_v, sem).wait()
    for g in range(B_PER_W // 16):
      v = g_v[pl.ds(g * 16, 16)]
      r = 1.0 / (1.0 + jnp.exp(-v))
      g_v[pl.ds(g * 16, 16)] = r
    pltpu.sync_copy(g_v, out_hbm.at[pl.ds(base, B_PER_W)])

  return select_k


_select = _make_select()


@jax.jit
def kernel(item_indices, emb_table, W, b):
  idx0 = (item_indices - 1).astype(jnp.int32)
  s = _matvec_stage(emb_table.T, W, b)
  return _select(idx0, s)
